# baseline (speedup 1.0000x reference)
import functools

import jax
import jax.numpy as jnp
from jax import lax
from jax.experimental import pallas as pl
from jax.experimental.pallas import tpu as pltpu
from jax.experimental.pallas import tpu_sc as plsc

BATCH = 16384
SEQ_LEN = 50
SEQ_PAD = 56
EMBED_DIM = 64
DIM_PAD = 128

_info = plsc.get_sparse_core_info()
NUM_WORKERS = _info.num_cores * _info.num_subcores
B_PER_WORKER = BATCH // NUM_WORKERS
CHB = 2
NUM_CHUNKS = B_PER_WORKER // CHB
CHUNK_ROWS = CHB * SEQ_PAD
NBUF = 4
NUM_GROUPS = NUM_CHUNKS // NBUF

_mesh = plsc.VectorSubcoreMesh(core_axis_name="c", subcore_axis_name="s")


@functools.partial(
    pl.kernel,
    mesh=_mesh,
    out_type=jax.ShapeDtypeStruct((BATCH, SEQ_PAD, DIM_PAD), jnp.float32),
    scratch_types=[
        pltpu.VMEM((B_PER_WORKER * SEQ_PAD,), jnp.int32),
        pltpu.VMEM((NBUF, CHUNK_ROWS, EMBED_DIM), jnp.float32),
        [pltpu.SemaphoreType.DMA] * NBUF,
        [pltpu.SemaphoreType.DMA] * NBUF,
    ],
    compiler_params=pltpu.CompilerParams(use_tc_tiling_on_sc=False),
)
def _gather_kernel(table_hbm, idx_hbm, out_hbm, idx_v, rows_v, gsems, ssems):
    wid = lax.axis_index("s") * _info.num_cores + lax.axis_index("c")
    b_base = wid * B_PER_WORKER
    pltpu.sync_copy(
        idx_hbm.at[pl.ds(b_base * SEQ_PAD, B_PER_WORKER * SEQ_PAD)], idx_v
    )

    def start_gather(nb, c):
        pltpu.async_copy(
            table_hbm.at[idx_v.at[pl.ds(c * CHUNK_ROWS, CHUNK_ROWS)]],
            rows_v.at[nb],
            gsems[nb],
        )

    def wait_gather(nb, c):
        pltpu.make_async_copy(
            table_hbm.at[idx_v.at[pl.ds(c * CHUNK_ROWS, CHUNK_ROWS)]],
            rows_v.at[nb],
            gsems[nb],
        ).wait()

    def start_store(nb, c):
        for k in range(CHB):
            pltpu.async_copy(
                rows_v.at[nb, pl.ds(k * SEQ_PAD, SEQ_PAD)],
                out_hbm.at[b_base + c * CHB + k, :, pl.ds(0, EMBED_DIM)],
                ssems[nb],
            )

    def wait_store(nb, c):
        for k in range(CHB):
            pltpu.make_async_copy(
                rows_v.at[nb, pl.ds(k * SEQ_PAD, SEQ_PAD)],
                out_hbm.at[b_base + c * CHB + k, :, pl.ds(0, EMBED_DIM)],
                ssems[nb],
            ).wait()

    for nb in range(NBUF):
        start_gather(nb, nb)

    def group_body(g, carry):
        for nb in range(NBUF):
            c = g * NBUF + nb
            wait_gather(nb, c)
            start_store(nb, c)
            cn = c + NBUF

            @pl.when(cn < NUM_CHUNKS)
            def _():
                wait_store(nb, c)
                start_gather(nb, cn)

        return carry

    lax.fori_loop(0, NUM_GROUPS, group_body, 0)

    for nb in range(NBUF):
        wait_store(nb, NUM_CHUNKS - NBUF + nb)


def kernel(indices, table):
    idx_p = jnp.pad(indices.astype(jnp.int32), ((0, 0), (0, SEQ_PAD - SEQ_LEN)))
    out_p = _gather_kernel(table, idx_p.reshape(-1))
    return out_p[:, :SEQ_LEN, :EMBED_DIM]


# speedup vs baseline: 3.3319x; 3.3319x over previous
import functools

import jax
import jax.numpy as jnp
from jax import lax
from jax.experimental import pallas as pl
from jax.experimental.pallas import tpu as pltpu
from jax.experimental.pallas import tpu_sc as plsc

BATCH = 16384
SEQ_LEN = 50
SEQ_PAD = 56
EMBED_DIM = 64
N = BATCH * SEQ_LEN
OUT_ROWS = BATCH * SEQ_PAD * 2

_info = plsc.get_sparse_core_info()
NUM_WORKERS = _info.num_cores * _info.num_subcores
PER_WORKER = N // NUM_WORKERS
CHUNK = 128
NUM_CHUNKS = PER_WORKER // CHUNK
NBUF = 4
NUM_GROUPS = NUM_CHUNKS // NBUF

_mesh = plsc.VectorSubcoreMesh(core_axis_name="c", subcore_axis_name="s")


@functools.partial(
    pl.kernel,
    mesh=_mesh,
    out_type=jax.ShapeDtypeStruct((OUT_ROWS, EMBED_DIM), jnp.float32),
    scratch_types=[
        pltpu.VMEM((NUM_CHUNKS, CHUNK), jnp.int32),
        pltpu.VMEM((NUM_CHUNKS, CHUNK), jnp.int32),
        pltpu.VMEM((NBUF, CHUNK, EMBED_DIM), jnp.float32),
        [pltpu.SemaphoreType.DMA] * NBUF,
        [pltpu.SemaphoreType.DMA] * NBUF,
    ],
    compiler_params=pltpu.CompilerParams(use_tc_tiling_on_sc=False),
)
def _gather_kernel(table_hbm, idx_hbm, pos_hbm, out_hbm, idx_v, pos_v, rows_v,
                   gsems, ssems):
    wid = lax.axis_index("s") * _info.num_cores + lax.axis_index("c")
    row_base = wid * NUM_CHUNKS
    pltpu.sync_copy(idx_hbm.at[pl.ds(row_base, NUM_CHUNKS)], idx_v)
    pltpu.sync_copy(pos_hbm.at[pl.ds(row_base, NUM_CHUNKS)], pos_v)

    def start_gather(nb, c):
        pltpu.async_copy(table_hbm.at[idx_v.at[c]], rows_v.at[nb], gsems[nb])

    def wait_gather(nb, c):
        pltpu.make_async_copy(
            table_hbm.at[idx_v.at[c]], rows_v.at[nb], gsems[nb]
        ).wait()

    def start_store(nb, c):
        pltpu.async_copy(rows_v.at[nb], out_hbm.at[pos_v.at[c]], ssems[nb])

    def wait_store(nb, c):
        pltpu.make_async_copy(
            rows_v.at[nb], out_hbm.at[pos_v.at[c]], ssems[nb]
        ).wait()

    for nb in range(NBUF):
        start_gather(nb, nb)

    def group_body(g, carry):
        for nb in range(NBUF):
            c = g * NBUF + nb
            wait_gather(nb, c)
            start_store(nb, c)
            cn = c + NBUF

            @pl.when(cn < NUM_CHUNKS)
            def _():
                wait_store(nb, c)
                start_gather(nb, cn)

        return carry

    lax.fori_loop(0, NUM_GROUPS, group_body, 0)

    for nb in range(NBUF):
        wait_store(nb, NUM_CHUNKS - NBUF + nb)


def kernel(indices, table):
    idx2 = indices.reshape(-1).astype(jnp.int32).reshape(N // CHUNK, CHUNK)
    flat = jnp.arange(N, dtype=jnp.int32)
    pos = 2 * (SEQ_PAD * (flat // SEQ_LEN) + flat % SEQ_LEN)
    pos2 = pos.reshape(N // CHUNK, CHUNK)
    out_flat = _gather_kernel(table, idx2, pos2)
    out_p = out_flat.reshape(BATCH, SEQ_PAD, 2 * EMBED_DIM)
    return out_p[:, :SEQ_LEN, :EMBED_DIM]


# speedup vs baseline: 3.3337x; 1.0005x over previous
import functools

import jax
import jax.numpy as jnp
from jax import lax
from jax.experimental import pallas as pl
from jax.experimental.pallas import tpu as pltpu
from jax.experimental.pallas import tpu_sc as plsc

BATCH = 16384
SEQ_LEN = 50
SEQ_PAD = 56
EMBED_DIM = 64
N = BATCH * SEQ_LEN
OUT_ROWS = BATCH * SEQ_PAD * 2

_info = plsc.get_sparse_core_info()
NUM_WORKERS = _info.num_cores * _info.num_subcores
PER_WORKER = N // NUM_WORKERS
CHUNK = 128
NUM_CHUNKS = PER_WORKER // CHUNK
NBUF = 8
NUM_GROUPS = NUM_CHUNKS // NBUF

_mesh = plsc.VectorSubcoreMesh(core_axis_name="c", subcore_axis_name="s")


@functools.partial(
    pl.kernel,
    mesh=_mesh,
    out_type=jax.ShapeDtypeStruct((OUT_ROWS, EMBED_DIM), jnp.float32),
    scratch_types=[
        pltpu.VMEM((NUM_CHUNKS, CHUNK), jnp.int32),
        pltpu.VMEM((NUM_CHUNKS, CHUNK), jnp.int32),
        pltpu.VMEM((NBUF, CHUNK, EMBED_DIM), jnp.float32),
        [pltpu.SemaphoreType.DMA] * NBUF,
        [pltpu.SemaphoreType.DMA] * NBUF,
    ],
    compiler_params=pltpu.CompilerParams(use_tc_tiling_on_sc=False),
)
def _gather_kernel(table_hbm, idx_hbm, pos_hbm, out_hbm, idx_v, pos_v, rows_v,
                   gsems, ssems):
    wid = lax.axis_index("s") * _info.num_cores + lax.axis_index("c")
    row_base = wid * NUM_CHUNKS
    pltpu.sync_copy(idx_hbm.at[pl.ds(row_base, NUM_CHUNKS)], idx_v)
    pltpu.sync_copy(pos_hbm.at[pl.ds(row_base, NUM_CHUNKS)], pos_v)

    def start_gather(nb, c):
        pltpu.async_copy(table_hbm.at[idx_v.at[c]], rows_v.at[nb], gsems[nb])

    def wait_gather(nb, c):
        pltpu.make_async_copy(
            table_hbm.at[idx_v.at[c]], rows_v.at[nb], gsems[nb]
        ).wait()

    def start_store(nb, c):
        pltpu.async_copy(rows_v.at[nb], out_hbm.at[pos_v.at[c]], ssems[nb])

    def wait_store(nb, c):
        pltpu.make_async_copy(
            rows_v.at[nb], out_hbm.at[pos_v.at[c]], ssems[nb]
        ).wait()

    for nb in range(NBUF):
        start_gather(nb, nb)

    def group_body(g, carry):
        for nb in range(NBUF):
            c = g * NBUF + nb
            wait_gather(nb, c)
            start_store(nb, c)
            cn = c + NBUF

            @pl.when(cn < NUM_CHUNKS)
            def _():
                wait_store(nb, c)
                start_gather(nb, cn)

        return carry

    lax.fori_loop(0, NUM_GROUPS, group_body, 0)

    for nb in range(NBUF):
        wait_store(nb, NUM_CHUNKS - NBUF + nb)


def kernel(indices, table):
    idx2 = indices.reshape(-1).astype(jnp.int32).reshape(N // CHUNK, CHUNK)
    flat = jnp.arange(N, dtype=jnp.int32)
    pos = 2 * (SEQ_PAD * (flat // SEQ_LEN) + flat % SEQ_LEN)
    pos2 = pos.reshape(N // CHUNK, CHUNK)
    out_flat = _gather_kernel(table, idx2, pos2)
    out_p = out_flat.reshape(BATCH, SEQ_PAD, 2 * EMBED_DIM)
    return out_p[:, :SEQ_LEN, :EMBED_DIM]
